# Initial kernel scaffold; baseline (speedup 1.0000x reference)
import jax
import jax.numpy as jnp
from jax.experimental import pallas as pl


def kernel(inputs, edge_index, W1_self, W1_neigh, b1, W2_self, W2_neigh, b2):
    raise NotImplementedError("write your pallas kernel here")


# speedup vs baseline: 3.5074x; 3.5074x over previous
import functools

import jax
import jax.numpy as jnp
from jax import lax
from jax.experimental import pallas as pl
from jax.experimental.pallas import tpu as pltpu
from jax.experimental.pallas import tpu_sc as plsc

NC = 2
NS = 16
CHUNK = 128


def _segsum(x, src_p, dst_p, npad):
    n, d = x.shape
    epad = src_p.shape[0]
    ch = epad // (NC * NS * CHUNK)
    rps = npad // NS
    kc = rps // CHUNK

    mesh = plsc.VectorSubcoreMesh(core_axis_name="c", subcore_axis_name="s")

    @functools.partial(
        pl.kernel,
        out_type=jax.ShapeDtypeStruct((NC * npad, d), jnp.float32),
        mesh=mesh,
        scratch_types=[
            pltpu.VMEM_SHARED((npad, d), jnp.float32),
            pltpu.VMEM((CHUNK,), jnp.int32),
            pltpu.VMEM((CHUNK,), jnp.int32),
            pltpu.VMEM((CHUNK, d), jnp.float32),
            pltpu.SemaphoreType.DMA,
        ],
    )
    def seg(x_hbm, src_hbm, dst_hbm, p_hbm, acc, isrc, idst, rows, sem):
        cid = lax.axis_index("c")
        sid = lax.axis_index("s")

        def fill_body(i, carry):
            for j in range(d // 16):
                rows[i, pl.ds(j * 16, 16)] = jnp.zeros((16,), jnp.float32)
            return carry
        lax.fori_loop(0, CHUNK, fill_body, 0)

        for k in range(kc):
            pltpu.sync_copy(rows, acc.at[pl.ds(sid * rps + k * CHUNK, CHUNK)])
        plsc.subcore_barrier()

        wid = cid * NS + sid
        ebase = wid * ch * CHUNK

        def chunk_body(i, carry):
            b = ebase + i * CHUNK
            pltpu.sync_copy(src_hbm.at[pl.ds(b, CHUNK)], isrc)
            pltpu.sync_copy(dst_hbm.at[pl.ds(b, CHUNK)], idst)
            pltpu.async_copy(x_hbm.at[isrc], rows, sem).wait()
            pltpu.sync_copy(rows, acc.at[idst], add=True)
            return carry
        lax.fori_loop(0, ch, chunk_body, 0)

        plsc.subcore_barrier()
        off = sid * rps
        pltpu.sync_copy(acc.at[pl.ds(off, rps)],
                        p_hbm.at[pl.ds(cid * npad + off, rps)])

    return seg(x, src_p, dst_p).reshape(NC, npad, d)


def _degsum(dst_p, npad, d):
    epad = dst_p.shape[0]
    ch = epad // (NC * NS * CHUNK)
    rps = npad // NS
    kc = rps // CHUNK

    mesh = plsc.VectorSubcoreMesh(core_axis_name="c", subcore_axis_name="s")

    @functools.partial(
        pl.kernel,
        out_type=jax.ShapeDtypeStruct((NC * npad, d), jnp.float32),
        mesh=mesh,
        scratch_types=[
            pltpu.VMEM_SHARED((npad, d), jnp.float32),
            pltpu.VMEM((CHUNK,), jnp.int32),
            pltpu.VMEM((CHUNK, d), jnp.float32),
        ],
    )
    def deg(dst_hbm, g_hbm, dacc, idst, ones):
        cid = lax.axis_index("c")
        sid = lax.axis_index("s")

        def fill_body(i, carry):
            for j in range(d // 16):
                ones[i, pl.ds(j * 16, 16)] = jnp.zeros((16,), jnp.float32)
            return carry
        lax.fori_loop(0, CHUNK, fill_body, 0)

        for k in range(kc):
            pltpu.sync_copy(ones, dacc.at[pl.ds(sid * rps + k * CHUNK, CHUNK)])

        def ones_body(i, carry):
            for j in range(d // 16):
                ones[i, pl.ds(j * 16, 16)] = jnp.ones((16,), jnp.float32)
            return carry
        lax.fori_loop(0, CHUNK, ones_body, 0)
        plsc.subcore_barrier()

        wid = cid * NS + sid
        ebase = wid * ch * CHUNK

        def chunk_body(i, carry):
            b = ebase + i * CHUNK
            pltpu.sync_copy(dst_hbm.at[pl.ds(b, CHUNK)], idst)
            pltpu.sync_copy(ones, dacc.at[idst], add=True)
            return carry
        lax.fori_loop(0, ch, chunk_body, 0)

        plsc.subcore_barrier()
        off = sid * rps
        pltpu.sync_copy(dacc.at[pl.ds(off, rps)],
                        g_hbm.at[pl.ds(cid * npad + off, rps)])

    return deg(dst_p).reshape(NC, npad, d)


def _combine(x, P, G, Ws, Wn, b2d, relu):
    n, d = x.shape
    blk = 400
    grid = n // blk

    def body(x_ref, p_ref, g_ref, ws_ref, wn_ref, b_ref, o_ref):
        deg = g_ref[0, :, 0:1] + g_ref[1, :, 0:1]
        hn = (p_ref[0] + p_ref[1]) / jnp.maximum(deg, 1.0)
        r = (jnp.dot(x_ref[...], ws_ref[...],
                     preferred_element_type=jnp.float32)
             + jnp.dot(hn, wn_ref[...], preferred_element_type=jnp.float32)
             + b_ref[...])
        o_ref[...] = jnp.maximum(r, 0.0) if relu else r

    return pl.pallas_call(
        body,
        grid=(grid,),
        in_specs=[
            pl.BlockSpec((blk, d), lambda i: (i, 0)),
            pl.BlockSpec((NC, blk, d), lambda i: (0, i, 0)),
            pl.BlockSpec((NC, blk, d), lambda i: (0, i, 0)),
            pl.BlockSpec((d, d), lambda i: (0, 0)),
            pl.BlockSpec((d, d), lambda i: (0, 0)),
            pl.BlockSpec((1, d), lambda i: (0, 0)),
        ],
        out_specs=pl.BlockSpec((blk, d), lambda i: (i, 0)),
        out_shape=jax.ShapeDtypeStruct((n, d), jnp.float32),
    )(x, P, G, Ws, Wn, b2d)


def kernel(inputs, edge_index, W1_self, W1_neigh, b1, W2_self, W2_neigh, b2):
    x = inputs
    n, d = x.shape
    e = edge_index.shape[1]
    src = edge_index[0].astype(jnp.int32)
    dst = edge_index[1].astype(jnp.int32)

    estep = NC * NS * CHUNK
    epad = ((e + estep - 1) // estep) * estep
    nstep = NS * CHUNK
    npad = ((n + 1 + nstep - 1) // nstep) * nstep

    pad = epad - e
    src_p = jnp.concatenate([src, jnp.zeros((pad,), jnp.int32)])
    dst_p = jnp.concatenate([dst, jnp.full((pad,), npad - 1, jnp.int32)])

    b1_2d = b1.reshape(1, d)
    b2_2d = b2.reshape(1, d)

    G = _degsum(dst_p, npad, d)
    P1 = _segsum(x, src_p, dst_p, npad)
    h1 = _combine(x, P1, G, W1_self, W1_neigh, b1_2d, relu=True)
    P2 = _segsum(h1, src_p, dst_p, npad)
    out = _combine(h1, P2, G, W2_self, W2_neigh, b2_2d, relu=False)
    return out


# speedup vs baseline: 4.0656x; 1.1592x over previous
import functools

import jax
import jax.numpy as jnp
from jax import lax
from jax.experimental import pallas as pl
from jax.experimental.pallas import tpu as pltpu
from jax.experimental.pallas import tpu_sc as plsc

NC = 2
NS = 16
CHUNK = 128


def _segsum(x, src_p, dst_p, npad):
    n, d = x.shape
    epad = src_p.shape[0]
    ch = epad // (NC * NS * CHUNK)
    rps = npad // NS
    kc = rps // CHUNK

    mesh = plsc.VectorSubcoreMesh(core_axis_name="c", subcore_axis_name="s")

    @functools.partial(
        pl.kernel,
        out_type=jax.ShapeDtypeStruct((NC * npad, d), jnp.float32),
        mesh=mesh,
        scratch_types=[
            pltpu.VMEM_SHARED((npad, d), jnp.float32),
            pltpu.VMEM((CHUNK,), jnp.int32),
            pltpu.VMEM((CHUNK,), jnp.int32),
            pltpu.VMEM((CHUNK,), jnp.int32),
            pltpu.VMEM((CHUNK,), jnp.int32),
            pltpu.VMEM((CHUNK, d), jnp.float32),
            pltpu.VMEM((CHUNK, d), jnp.float32),
            pltpu.SemaphoreType.DMA,
            pltpu.SemaphoreType.DMA,
        ],
    )
    def seg(x_hbm, src_hbm, dst_hbm, p_hbm,
            acc, isrc_a, idst_a, isrc_b, idst_b, rows_a, rows_b,
            sem_a, sem_b):
        cid = lax.axis_index("c")
        sid = lax.axis_index("s")

        def fill_body(i, carry):
            for j in range(d // 16):
                rows_a[i, pl.ds(j * 16, 16)] = jnp.zeros((16,), jnp.float32)
            return carry
        lax.fori_loop(0, CHUNK, fill_body, 0)

        for k in range(kc):
            pltpu.sync_copy(rows_a, acc.at[pl.ds(sid * rps + k * CHUNK, CHUNK)])
        plsc.subcore_barrier()

        wid = cid * NS + sid
        ebase = wid * ch * CHUNK

        def pair_body(i, carry):
            b = ebase + 2 * i * CHUNK
            pltpu.sync_copy(src_hbm.at[pl.ds(b, CHUNK)], isrc_a)
            pltpu.sync_copy(dst_hbm.at[pl.ds(b, CHUNK)], idst_a)
            cp_a = pltpu.async_copy(x_hbm.at[isrc_a], rows_a, sem_a)
            pltpu.sync_copy(src_hbm.at[pl.ds(b + CHUNK, CHUNK)], isrc_b)
            pltpu.sync_copy(dst_hbm.at[pl.ds(b + CHUNK, CHUNK)], idst_b)
            cp_b = pltpu.async_copy(x_hbm.at[isrc_b], rows_b, sem_b)
            cp_a.wait()
            pltpu.sync_copy(rows_a, acc.at[idst_a], add=True)
            cp_b.wait()
            pltpu.sync_copy(rows_b, acc.at[idst_b], add=True)
            return carry
        lax.fori_loop(0, ch // 2, pair_body, 0)

        if ch % 2:
            b = ebase + (ch - 1) * CHUNK
            pltpu.sync_copy(src_hbm.at[pl.ds(b, CHUNK)], isrc_a)
            pltpu.sync_copy(dst_hbm.at[pl.ds(b, CHUNK)], idst_a)
            pltpu.async_copy(x_hbm.at[isrc_a], rows_a, sem_a).wait()
            pltpu.sync_copy(rows_a, acc.at[idst_a], add=True)

        plsc.subcore_barrier()
        off = sid * rps
        pltpu.sync_copy(acc.at[pl.ds(off, rps)],
                        p_hbm.at[pl.ds(cid * npad + off, rps)])

    return seg(x, src_p, dst_p).reshape(NC, npad, d)


def _degsum(dst_p, npad, d):
    epad = dst_p.shape[0]
    ch = epad // (NC * NS * CHUNK)
    rps = npad // NS
    kc = rps // CHUNK

    mesh = plsc.VectorSubcoreMesh(core_axis_name="c", subcore_axis_name="s")

    @functools.partial(
        pl.kernel,
        out_type=jax.ShapeDtypeStruct((NC * npad, d), jnp.float32),
        mesh=mesh,
        scratch_types=[
            pltpu.VMEM_SHARED((npad, d), jnp.float32),
            pltpu.VMEM((CHUNK,), jnp.int32),
            pltpu.VMEM((CHUNK, d), jnp.float32),
        ],
    )
    def deg(dst_hbm, g_hbm, dacc, idst, ones):
        cid = lax.axis_index("c")
        sid = lax.axis_index("s")

        def fill_body(i, carry):
            for j in range(d // 16):
                ones[i, pl.ds(j * 16, 16)] = jnp.zeros((16,), jnp.float32)
            return carry
        lax.fori_loop(0, CHUNK, fill_body, 0)

        for k in range(kc):
            pltpu.sync_copy(ones, dacc.at[pl.ds(sid * rps + k * CHUNK, CHUNK)])

        def ones_body(i, carry):
            for j in range(d // 16):
                ones[i, pl.ds(j * 16, 16)] = jnp.ones((16,), jnp.float32)
            return carry
        lax.fori_loop(0, CHUNK, ones_body, 0)
        plsc.subcore_barrier()

        wid = cid * NS + sid
        ebase = wid * ch * CHUNK

        def chunk_body(i, carry):
            b = ebase + i * CHUNK
            pltpu.sync_copy(dst_hbm.at[pl.ds(b, CHUNK)], idst)
            pltpu.sync_copy(ones, dacc.at[idst], add=True)
            return carry
        lax.fori_loop(0, ch, chunk_body, 0)

        plsc.subcore_barrier()
        off = sid * rps
        pltpu.sync_copy(dacc.at[pl.ds(off, rps)],
                        g_hbm.at[pl.ds(cid * npad + off, rps)])

    return deg(dst_p).reshape(NC, npad, d)


def _combine(x, P, G, Ws, Wn, b2d, relu):
    n, d = x.shape
    blk = 400
    grid = n // blk

    def body(x_ref, p_ref, g_ref, ws_ref, wn_ref, b_ref, o_ref):
        deg = g_ref[0, :, 0:1] + g_ref[1, :, 0:1]
        hn = (p_ref[0] + p_ref[1]) / jnp.maximum(deg, 1.0)
        r = (jnp.dot(x_ref[...], ws_ref[...],
                     preferred_element_type=jnp.float32)
             + jnp.dot(hn, wn_ref[...], preferred_element_type=jnp.float32)
             + b_ref[...])
        o_ref[...] = jnp.maximum(r, 0.0) if relu else r

    return pl.pallas_call(
        body,
        grid=(grid,),
        in_specs=[
            pl.BlockSpec((blk, d), lambda i: (i, 0)),
            pl.BlockSpec((NC, blk, d), lambda i: (0, i, 0)),
            pl.BlockSpec((NC, blk, d), lambda i: (0, i, 0)),
            pl.BlockSpec((d, d), lambda i: (0, 0)),
            pl.BlockSpec((d, d), lambda i: (0, 0)),
            pl.BlockSpec((1, d), lambda i: (0, 0)),
        ],
        out_specs=pl.BlockSpec((blk, d), lambda i: (i, 0)),
        out_shape=jax.ShapeDtypeStruct((n, d), jnp.float32),
    )(x, P, G, Ws, Wn, b2d)


def kernel(inputs, edge_index, W1_self, W1_neigh, b1, W2_self, W2_neigh, b2):
    x = inputs
    n, d = x.shape
    e = edge_index.shape[1]
    src = edge_index[0].astype(jnp.int32)
    dst = edge_index[1].astype(jnp.int32)

    estep = NC * NS * CHUNK
    epad = ((e + estep - 1) // estep) * estep
    nstep = NS * CHUNK
    npad = ((n + 1 + nstep - 1) // nstep) * nstep

    pad = epad - e
    src_p = jnp.concatenate([src, jnp.zeros((pad,), jnp.int32)])
    dst_p = jnp.concatenate([dst, jnp.full((pad,), npad - 1, jnp.int32)])

    b1_2d = b1.reshape(1, d)
    b2_2d = b2.reshape(1, d)

    G = _degsum(dst_p, npad, d)
    P1 = _segsum(x, src_p, dst_p, npad)
    h1 = _combine(x, P1, G, W1_self, W1_neigh, b1_2d, relu=True)
    P2 = _segsum(h1, src_p, dst_p, npad)
    out = _combine(h1, P2, G, W2_self, W2_neigh, b2_2d, relu=False)
    return out
